# baseline (device time: 1011593 ns/iter reference)
import jax
import jax.numpy as jnp
from jax import lax
from jax.experimental import pallas as pl
from jax.experimental.pallas import tpu as pltpu

N_DEV = 32


def kernel(x, w_mat):
    m_per, k = x.shape
    _, n_per = w_mat.shape
    m_tot = N_DEV * m_per

    def _gemm_block(xin, w):
        y = jnp.dot(xin, w, preferred_element_type=jnp.float32,
                    precision=lax.Precision.HIGHEST)
        return jnp.maximum(y, 0.0)

    def body(x_ref, w_ref, out_ref, comm_ref, asend_ref, arecv_ref,
             send_sems, recv_sems, credit_sems, amax_send_sems,
             amax_recv_sems):
        me = lax.axis_index("i")
        left = (me - 1) % N_DEV
        right = (me + 1) % N_DEV

        barrier_sem = pltpu.get_barrier_semaphore()
        pl.semaphore_signal(barrier_sem, inc=1, device_id=(left,),
                            device_id_type=pl.DeviceIdType.MESH)
        pl.semaphore_signal(barrier_sem, inc=1, device_id=(right,),
                            device_id_type=pl.DeviceIdType.MESH)
        pl.semaphore_wait(barrier_sem, 2)

        rdma0 = pltpu.make_async_remote_copy(
            src_ref=x_ref, dst_ref=comm_ref.at[0],
            send_sem=send_sems.at[0], recv_sem=recv_sems.at[0],
            device_id=(right,), device_id_type=pl.DeviceIdType.MESH)
        rdma0.start()
        blk = _gemm_block(x_ref[...], w_ref[...])
        out_ref[pl.ds(me * m_per, m_per), :] = blk
        amax = jnp.max(blk)
        rdma0.wait()
        blk = _gemm_block(comm_ref[0], w_ref[...])
        out_ref[pl.ds(((me - 1) % N_DEV) * m_per, m_per), :] = blk
        amax = jnp.maximum(amax, jnp.max(blk))

        def hop(h, amax):
            s = h % 2
            sp = (h - 1) % 2

            @pl.when(h >= 2)
            def _():
                pl.semaphore_wait(credit_sems.at[s], 1)

            rdma = pltpu.make_async_remote_copy(
                src_ref=comm_ref.at[sp], dst_ref=comm_ref.at[s],
                send_sem=send_sems.at[sp], recv_sem=recv_sems.at[s],
                device_id=(right,), device_id_type=pl.DeviceIdType.MESH)
            rdma.start()
            rdma.wait()

            origin = (me - h - 1) % N_DEV
            blk = _gemm_block(comm_ref[s], w_ref[...])
            out_ref[pl.ds(origin * m_per, m_per), :] = blk

            @pl.when(h <= N_DEV - 3)
            def _():
                pl.semaphore_signal(credit_sems.at[sp], inc=1,
                                    device_id=(left,),
                                    device_id_type=pl.DeviceIdType.MESH)

            return jnp.maximum(amax, jnp.max(blk))

        amax = lax.fori_loop(1, N_DEV - 1, hop, amax)

        for r in range(5):
            partner = me ^ (1 << r)
            asend_ref[...] = jnp.full((1, 128), amax, jnp.float32)
            rdma = pltpu.make_async_remote_copy(
                src_ref=asend_ref, dst_ref=arecv_ref.at[r],
                send_sem=amax_send_sems.at[r],
                recv_sem=amax_recv_sems.at[r],
                device_id=(partner,), device_id_type=pl.DeviceIdType.MESH)
            rdma.start()
            rdma.wait()
            amax = jnp.maximum(amax, jnp.max(arecv_ref[r]))

        scale = amax / 448.0
        a = jnp.minimum(out_ref[...] / scale, 448.0)
        u = lax.bitcast_convert_type(a, jnp.int32)
        lsb = lax.shift_right_logical(u, 20) & 1
        ur = lax.shift_left(
            lax.shift_right_logical(u + 0x7FFFF + lsb, 20), 20)
        q = lax.bitcast_convert_type(ur, jnp.float32)
        out_ref[...] = q * scale

    return pl.pallas_call(
        body,
        out_shape=jax.ShapeDtypeStruct((m_tot, n_per), jnp.float32),
        in_specs=[pl.BlockSpec(memory_space=pltpu.VMEM),
                  pl.BlockSpec(memory_space=pltpu.VMEM)],
        out_specs=pl.BlockSpec(memory_space=pltpu.VMEM),
        scratch_shapes=[
            pltpu.VMEM((2, m_per, k), jnp.float32),
            pltpu.VMEM((1, 128), jnp.float32),
            pltpu.VMEM((5, 1, 128), jnp.float32),
            pltpu.SemaphoreType.DMA((2,)),
            pltpu.SemaphoreType.DMA((2,)),
            pltpu.SemaphoreType.REGULAR((2,)),
            pltpu.SemaphoreType.DMA((5,)),
            pltpu.SemaphoreType.DMA((5,)),
        ],
        compiler_params=pltpu.CompilerParams(collective_id=0),
    )(x, w_mat)


# device time: 747261 ns/iter; 1.3537x vs baseline; 1.3537x over previous
import jax
import jax.numpy as jnp
from jax import lax
from jax.experimental import pallas as pl
from jax.experimental.pallas import tpu as pltpu

N_DEV = 32
H_F = 16
H_B = 15


def kernel(x, w_mat):
    m_per, k = x.shape
    _, n_per = w_mat.shape
    m_tot = N_DEV * m_per

    def body(x_ref, w_ref, out_ref, fcomm, bcomm, asend_ref, arecv_ref,
             fsend_sems, frecv_sems, fcredit, bsend_sems, brecv_sems,
             bcredit, amax_send_sems, amax_recv_sems):
        me = lax.axis_index("i")
        left = (me - 1) % N_DEV
        right = (me + 1) % N_DEV

        def gemm_store(xin, origin):
            blk = jnp.dot(xin, w_ref[...],
                          preferred_element_type=jnp.float32,
                          precision=lax.Precision.HIGHEST)
            blk = jnp.maximum(blk, 0.0)
            out_ref[pl.ds(origin * m_per, m_per), :] = blk
            return jnp.max(blk)

        def fdesc(h):
            src = x_ref if h == 0 else fcomm.at[(h - 1) % 2]
            return pltpu.make_async_remote_copy(
                src_ref=src, dst_ref=fcomm.at[h % 2],
                send_sem=fsend_sems.at[h % 2], recv_sem=frecv_sems.at[h % 2],
                device_id=(right,), device_id_type=pl.DeviceIdType.MESH)

        def bdesc(h):
            src = x_ref if h == 0 else bcomm.at[(h - 1) % 2]
            return pltpu.make_async_remote_copy(
                src_ref=src, dst_ref=bcomm.at[h % 2],
                send_sem=bsend_sems.at[h % 2], recv_sem=brecv_sems.at[h % 2],
                device_id=(left,), device_id_type=pl.DeviceIdType.MESH)

        barrier_sem = pltpu.get_barrier_semaphore()
        pl.semaphore_signal(barrier_sem, inc=1, device_id=(left,),
                            device_id_type=pl.DeviceIdType.MESH)
        pl.semaphore_signal(barrier_sem, inc=1, device_id=(right,),
                            device_id_type=pl.DeviceIdType.MESH)
        pl.semaphore_wait(barrier_sem, 2)

        f0 = fdesc(0)
        b0 = bdesc(0)
        f0.start()
        b0.start()
        amax = gemm_store(x_ref[...], me)
        f0.wait_send()
        b0.wait_send()

        for h in range(1, H_F):
            fr = fdesc(h - 1)
            fr.wait_recv()
            if h >= 2:
                pl.semaphore_wait(fcredit.at[h % 2], 1)
            fh = fdesc(h)
            fh.start()

            br = bdesc(h - 1)
            br.wait_recv()
            if 2 <= h <= H_B - 1:
                pl.semaphore_wait(bcredit.at[h % 2], 1)
            if h <= H_B - 1:
                bh = bdesc(h)
                bh.start()

            amax = jnp.maximum(amax, gemm_store(fcomm[(h - 1) % 2],
                                                (me - h) % N_DEV))
            amax = jnp.maximum(amax, gemm_store(bcomm[(h - 1) % 2],
                                                (me + h) % N_DEV))

            fh.wait_send()
            if 1 <= h <= H_F - 2:
                pl.semaphore_signal(fcredit.at[(h - 1) % 2], inc=1,
                                    device_id=(left,),
                                    device_id_type=pl.DeviceIdType.MESH)
            if h <= H_B - 1:
                bh.wait_send()
            if 1 <= h <= H_B - 2:
                pl.semaphore_signal(bcredit.at[(h - 1) % 2], inc=1,
                                    device_id=(right,),
                                    device_id_type=pl.DeviceIdType.MESH)

        fr = fdesc(H_F - 1)
        fr.wait_recv()
        amax = jnp.maximum(amax, gemm_store(fcomm[(H_F - 1) % 2],
                                            (me - H_F) % N_DEV))

        for r in range(5):
            partner = me ^ (1 << r)
            asend_ref[...] = jnp.full((1, 128), amax, jnp.float32)
            rdma = pltpu.make_async_remote_copy(
                src_ref=asend_ref, dst_ref=arecv_ref.at[r],
                send_sem=amax_send_sems.at[r],
                recv_sem=amax_recv_sems.at[r],
                device_id=(partner,), device_id_type=pl.DeviceIdType.MESH)
            rdma.start()
            rdma.wait()
            amax = jnp.maximum(amax, jnp.max(arecv_ref[r]))

        scale = amax / 448.0
        a = jnp.minimum(out_ref[...] / scale, 448.0)
        u = lax.bitcast_convert_type(a, jnp.int32)
        lsb = lax.shift_right_logical(u, 20) & 1
        ur = lax.shift_left(
            lax.shift_right_logical(u + 0x7FFFF + lsb, 20), 20)
        q = lax.bitcast_convert_type(ur, jnp.float32)
        out_ref[...] = q * scale

    return pl.pallas_call(
        body,
        out_shape=jax.ShapeDtypeStruct((m_tot, n_per), jnp.float32),
        in_specs=[pl.BlockSpec(memory_space=pltpu.VMEM),
                  pl.BlockSpec(memory_space=pltpu.VMEM)],
        out_specs=pl.BlockSpec(memory_space=pltpu.VMEM),
        scratch_shapes=[
            pltpu.VMEM((2, m_per, k), jnp.float32),
            pltpu.VMEM((2, m_per, k), jnp.float32),
            pltpu.VMEM((1, 128), jnp.float32),
            pltpu.VMEM((5, 1, 128), jnp.float32),
            pltpu.SemaphoreType.DMA((2,)),
            pltpu.SemaphoreType.DMA((2,)),
            pltpu.SemaphoreType.REGULAR((2,)),
            pltpu.SemaphoreType.DMA((2,)),
            pltpu.SemaphoreType.DMA((2,)),
            pltpu.SemaphoreType.REGULAR((2,)),
            pltpu.SemaphoreType.DMA((5,)),
            pltpu.SemaphoreType.DMA((5,)),
        ],
        compiler_params=pltpu.CompilerParams(collective_id=0),
    )(x, w_mat)


# device time: 419820 ns/iter; 2.4096x vs baseline; 1.7800x over previous
import numpy as np

import jax
import jax.numpy as jnp
from jax import lax
from jax.experimental import pallas as pl
from jax.experimental.pallas import tpu as pltpu

N_DEV = 32
H_F = 16
H_B = 15


def _snake_logical(x, y, z):
    return z * 8 + y * 2 + (x if y % 2 == 0 else 1 - x)


def _ham_cycle():
    v16 = [(0, 0), (0, 1), (0, 2), (0, 3), (1, 3), (1, 2), (1, 1),
           (2, 1), (2, 2), (2, 3), (3, 3), (3, 2), (3, 1), (3, 0),
           (2, 0), (1, 0)]
    ham = [_snake_logical(0, y, z) for (y, z) in v16]
    ham += [_snake_logical(1, y, z) for (y, z) in reversed(v16)]
    return np.array(ham, np.int32)


_HAM = _ham_cycle()
_POS = np.zeros(N_DEV, np.int32)
_POS[_HAM] = np.arange(N_DEV, dtype=np.int32)


def kernel(x, w_mat):
    m_per, k = x.shape
    _, n_per = w_mat.shape
    m_tot = N_DEV * m_per

    me = lax.axis_index("i")
    ham = jnp.asarray(_HAM)
    p = jnp.asarray(_POS)[me]
    nxt = ham[(p + 1) % N_DEV]
    prv = ham[(p - 1) % N_DEV]
    fwd_origins = ham[(p - 1 - jnp.arange(H_F)) % N_DEV]
    bwd_origins = ham[(p + 1 + jnp.arange(H_B)) % N_DEV]
    ids = jnp.concatenate(
        [jnp.stack([nxt, prv]), fwd_origins, bwd_origins]).astype(jnp.int32)

    def body(ids_ref, x_ref, w_ref, out_ref, fcomm, bcomm, asend_ref,
             arecv_ref, fsend_sems, frecv_sems, fcredit, bsend_sems,
             brecv_sems, bcredit, amax_send_sems, amax_recv_sems):
        me = lax.axis_index("i")
        nxt = ids_ref[0]
        prv = ids_ref[1]

        def gemm_store(xin, origin):
            blk = jnp.dot(xin, w_ref[...],
                          preferred_element_type=jnp.float32,
                          precision=lax.Precision.HIGHEST)
            blk = jnp.maximum(blk, 0.0)
            out_ref[pl.ds(origin * m_per, m_per), :] = blk
            return jnp.max(blk)

        def fdesc(h):
            src = x_ref if h == 0 else fcomm.at[(h - 1) % 2]
            return pltpu.make_async_remote_copy(
                src_ref=src, dst_ref=fcomm.at[h % 2],
                send_sem=fsend_sems.at[h % 2], recv_sem=frecv_sems.at[h % 2],
                device_id=(nxt,), device_id_type=pl.DeviceIdType.MESH)

        def bdesc(h):
            src = x_ref if h == 0 else bcomm.at[(h - 1) % 2]
            return pltpu.make_async_remote_copy(
                src_ref=src, dst_ref=bcomm.at[h % 2],
                send_sem=bsend_sems.at[h % 2], recv_sem=brecv_sems.at[h % 2],
                device_id=(prv,), device_id_type=pl.DeviceIdType.MESH)

        barrier_sem = pltpu.get_barrier_semaphore()
        pl.semaphore_signal(barrier_sem, inc=1, device_id=(prv,),
                            device_id_type=pl.DeviceIdType.MESH)
        pl.semaphore_signal(barrier_sem, inc=1, device_id=(nxt,),
                            device_id_type=pl.DeviceIdType.MESH)
        pl.semaphore_wait(barrier_sem, 2)

        f0 = fdesc(0)
        b0 = bdesc(0)
        f0.start()
        b0.start()
        amax = gemm_store(x_ref[...], me)
        f0.wait_send()
        b0.wait_send()

        for h in range(1, H_F):
            fr = fdesc(h - 1)
            fr.wait_recv()
            if h >= 2:
                pl.semaphore_wait(fcredit.at[h % 2], 1)
            fh = fdesc(h)
            fh.start()

            br = bdesc(h - 1)
            br.wait_recv()
            if 2 <= h <= H_B - 1:
                pl.semaphore_wait(bcredit.at[h % 2], 1)
            if h <= H_B - 1:
                bh = bdesc(h)
                bh.start()

            amax = jnp.maximum(amax, gemm_store(fcomm[(h - 1) % 2],
                                                ids_ref[2 + (h - 1)]))
            amax = jnp.maximum(amax, gemm_store(bcomm[(h - 1) % 2],
                                                ids_ref[2 + H_F + (h - 1)]))

            fh.wait_send()
            if 1 <= h <= H_F - 2:
                pl.semaphore_signal(fcredit.at[(h - 1) % 2], inc=1,
                                    device_id=(prv,),
                                    device_id_type=pl.DeviceIdType.MESH)
            if h <= H_B - 1:
                bh.wait_send()
            if 1 <= h <= H_B - 2:
                pl.semaphore_signal(bcredit.at[(h - 1) % 2], inc=1,
                                    device_id=(nxt,),
                                    device_id_type=pl.DeviceIdType.MESH)

        fr = fdesc(H_F - 1)
        fr.wait_recv()
        amax = jnp.maximum(amax, gemm_store(fcomm[(H_F - 1) % 2],
                                            ids_ref[2 + (H_F - 1)]))

        for r in range(5):
            partner = me ^ (1 << r)
            asend_ref[...] = jnp.full((1, 128), amax, jnp.float32)
            rdma = pltpu.make_async_remote_copy(
                src_ref=asend_ref, dst_ref=arecv_ref.at[r],
                send_sem=amax_send_sems.at[r],
                recv_sem=amax_recv_sems.at[r],
                device_id=(partner,), device_id_type=pl.DeviceIdType.MESH)
            rdma.start()
            rdma.wait()
            amax = jnp.maximum(amax, jnp.max(arecv_ref[r]))

        scale = amax / 448.0
        a = jnp.minimum(out_ref[...] / scale, 448.0)
        u = lax.bitcast_convert_type(a, jnp.int32)
        lsb = lax.shift_right_logical(u, 20) & 1
        ur = lax.shift_left(
            lax.shift_right_logical(u + 0x7FFFF + lsb, 20), 20)
        q = lax.bitcast_convert_type(ur, jnp.float32)
        out_ref[...] = q * scale

    return pl.pallas_call(
        body,
        out_shape=jax.ShapeDtypeStruct((m_tot, n_per), jnp.float32),
        in_specs=[pl.BlockSpec(memory_space=pltpu.SMEM),
                  pl.BlockSpec(memory_space=pltpu.VMEM),
                  pl.BlockSpec(memory_space=pltpu.VMEM)],
        out_specs=pl.BlockSpec(memory_space=pltpu.VMEM),
        scratch_shapes=[
            pltpu.VMEM((2, m_per, k), jnp.float32),
            pltpu.VMEM((2, m_per, k), jnp.float32),
            pltpu.VMEM((1, 128), jnp.float32),
            pltpu.VMEM((5, 1, 128), jnp.float32),
            pltpu.SemaphoreType.DMA((2,)),
            pltpu.SemaphoreType.DMA((2,)),
            pltpu.SemaphoreType.REGULAR((2,)),
            pltpu.SemaphoreType.DMA((2,)),
            pltpu.SemaphoreType.DMA((2,)),
            pltpu.SemaphoreType.REGULAR((2,)),
            pltpu.SemaphoreType.DMA((5,)),
            pltpu.SemaphoreType.DMA((5,)),
        ],
        compiler_params=pltpu.CompilerParams(collective_id=0),
    )(ids, x, w_mat)


# device time: 409631 ns/iter; 2.4695x vs baseline; 1.0249x over previous
import numpy as np

import jax
import jax.numpy as jnp
from jax import lax
from jax.experimental import pallas as pl
from jax.experimental.pallas import tpu as pltpu

N_DEV = 32
H_F = 16
H_B = 15


def _snake_logical(x, y, z):
    return z * 8 + y * 2 + (x if y % 2 == 0 else 1 - x)


def _ham_cycle():
    v16 = [(0, 0), (0, 1), (0, 2), (0, 3), (1, 3), (1, 2), (1, 1),
           (2, 1), (2, 2), (2, 3), (3, 3), (3, 2), (3, 1), (3, 0),
           (2, 0), (1, 0)]
    ham = [_snake_logical(0, y, z) for (y, z) in v16]
    ham += [_snake_logical(1, y, z) for (y, z) in reversed(v16)]
    return np.array(ham, np.int32)


_HAM = _ham_cycle()
_POS = np.zeros(N_DEV, np.int32)
_POS[_HAM] = np.arange(N_DEV, dtype=np.int32)


def kernel(x, w_mat):
    m_per, k = x.shape
    _, n_per = w_mat.shape
    m_tot = N_DEV * m_per
    kh = k // 2

    me = lax.axis_index("i")
    ham = jnp.asarray(_HAM)
    p = jnp.asarray(_POS)[me]
    nxt = ham[(p + 1) % N_DEV]
    prv = ham[(p - 1) % N_DEV]
    fwd_origins = ham[(p - 1 - jnp.arange(H_F)) % N_DEV]
    bwd_origins = ham[(p + 1 + jnp.arange(H_B)) % N_DEV]
    ids = jnp.concatenate(
        [jnp.stack([nxt, prv]), fwd_origins, bwd_origins]).astype(jnp.int32)

    def body(ids_ref, x_ref, w_ref, out_ref, fcomm, bcomm, asend_ref,
             arecv_ref, fsend_sems, frecv_sems, fcredit, bsend_sems,
             brecv_sems, bcredit, absend_sems, abrecv_sems):
        me = lax.axis_index("i")
        nxt = ids_ref[0]
        prv = ids_ref[1]

        def gemm_store2(sub0, sub1, origin):
            blk = (jnp.dot(sub0, w_ref[0:kh, :],
                           preferred_element_type=jnp.float32,
                           precision=lax.Precision.HIGHEST)
                   + jnp.dot(sub1, w_ref[kh:k, :],
                             preferred_element_type=jnp.float32,
                             precision=lax.Precision.HIGHEST))
            blk = jnp.maximum(blk, 0.0)
            out_ref[pl.ds(origin * m_per, m_per), :] = blk
            return jnp.max(blk)

        def fdesc(h, j):
            src = (x_ref.at[:, pl.ds(j * kh, kh)] if h == 0
                   else fcomm.at[(h - 1) % 2, j])
            return pltpu.make_async_remote_copy(
                src_ref=src, dst_ref=fcomm.at[h % 2, j],
                send_sem=fsend_sems.at[h % 2, j],
                recv_sem=frecv_sems.at[h % 2, j],
                device_id=(nxt,), device_id_type=pl.DeviceIdType.MESH)

        def bdesc(h, j):
            src = (x_ref.at[:, pl.ds(j * kh, kh)] if h == 0
                   else bcomm.at[(h - 1) % 2, j])
            return pltpu.make_async_remote_copy(
                src_ref=src, dst_ref=bcomm.at[h % 2, j],
                send_sem=bsend_sems.at[h % 2, j],
                recv_sem=brecv_sems.at[h % 2, j],
                device_id=(prv,), device_id_type=pl.DeviceIdType.MESH)

        barrier_sem = pltpu.get_barrier_semaphore()
        pl.semaphore_signal(barrier_sem, inc=1, device_id=(prv,),
                            device_id_type=pl.DeviceIdType.MESH)
        pl.semaphore_signal(barrier_sem, inc=1, device_id=(nxt,),
                            device_id_type=pl.DeviceIdType.MESH)
        pl.semaphore_wait(barrier_sem, 2)

        starts0 = [fdesc(0, 0), fdesc(0, 1), bdesc(0, 0), bdesc(0, 1)]
        for d in starts0:
            d.start()
        amax = gemm_store2(x_ref[:, 0:kh], x_ref[:, kh:k], me)
        for d in starts0:
            d.wait_send()

        for h in range(1, H_F):
            sp = (h - 1) % 2
            fr0 = fdesc(h - 1, 0)
            fr0.wait_recv()
            if h >= 2:
                pl.semaphore_wait(fcredit.at[h % 2], 1)
            fh0 = fdesc(h, 0)
            fh0.start()
            fr1 = fdesc(h - 1, 1)
            fr1.wait_recv()
            fh1 = fdesc(h, 1)
            fh1.start()

            br0 = bdesc(h - 1, 0)
            br0.wait_recv()
            if 2 <= h <= H_B - 1:
                pl.semaphore_wait(bcredit.at[h % 2], 1)
            if h <= H_B - 1:
                bh0 = bdesc(h, 0)
                bh0.start()
            br1 = bdesc(h - 1, 1)
            br1.wait_recv()
            if h <= H_B - 1:
                bh1 = bdesc(h, 1)
                bh1.start()

            amax = jnp.maximum(amax, gemm_store2(
                fcomm[sp, 0], fcomm[sp, 1], ids_ref[2 + (h - 1)]))
            amax = jnp.maximum(amax, gemm_store2(
                bcomm[sp, 0], bcomm[sp, 1], ids_ref[2 + H_F + (h - 1)]))

            fh0.wait_send()
            fh1.wait_send()
            if 1 <= h <= H_F - 2:
                pl.semaphore_signal(fcredit.at[sp], inc=1,
                                    device_id=(prv,),
                                    device_id_type=pl.DeviceIdType.MESH)
            if h <= H_B - 1:
                bh0.wait_send()
                bh1.wait_send()
            if 1 <= h <= H_B - 2:
                pl.semaphore_signal(bcredit.at[sp], inc=1,
                                    device_id=(nxt,),
                                    device_id_type=pl.DeviceIdType.MESH)

        spf = (H_F - 1) % 2
        fr0 = fdesc(H_F - 1, 0)
        fr0.wait_recv()
        fr1 = fdesc(H_F - 1, 1)
        fr1.wait_recv()
        amax = jnp.maximum(amax, gemm_store2(
            fcomm[spf, 0], fcomm[spf, 1], ids_ref[2 + (H_F - 1)]))

        asend_ref[...] = jnp.full((1, 128), amax, jnp.float32)
        sends = []
        for jj in range(1, N_DEV):
            tgt = (me + jj) % N_DEV
            d = pltpu.make_async_remote_copy(
                src_ref=asend_ref, dst_ref=arecv_ref.at[me],
                send_sem=absend_sems.at[jj],
                recv_sem=abrecv_sems.at[me],
                device_id=(tgt,), device_id_type=pl.DeviceIdType.MESH)
            d.start()
            sends.append(d)
        arecv_ref[pl.ds(me, 1), :, :] = jnp.full((1, 1, 128), amax,
                                                 jnp.float32)
        for jj in range(1, N_DEV):
            j = (me + jj) % N_DEV
            rw = pltpu.make_async_remote_copy(
                src_ref=asend_ref, dst_ref=arecv_ref.at[j],
                send_sem=absend_sems.at[0], recv_sem=abrecv_sems.at[j],
                device_id=(me,), device_id_type=pl.DeviceIdType.MESH)
            rw.wait_recv()
        for d in sends:
            d.wait_send()
        amax = jnp.max(arecv_ref[...])

        scale = amax / 448.0
        a = jnp.minimum(out_ref[...] / scale, 448.0)
        u = lax.bitcast_convert_type(a, jnp.int32)
        lsb = lax.shift_right_logical(u, 20) & 1
        ur = lax.shift_left(
            lax.shift_right_logical(u + 0x7FFFF + lsb, 20), 20)
        q = lax.bitcast_convert_type(ur, jnp.float32)
        out_ref[...] = q * scale

    return pl.pallas_call(
        body,
        out_shape=jax.ShapeDtypeStruct((m_tot, n_per), jnp.float32),
        in_specs=[pl.BlockSpec(memory_space=pltpu.SMEM),
                  pl.BlockSpec(memory_space=pltpu.VMEM),
                  pl.BlockSpec(memory_space=pltpu.VMEM)],
        out_specs=pl.BlockSpec(memory_space=pltpu.VMEM),
        scratch_shapes=[
            pltpu.VMEM((2, 2, m_per, kh), jnp.float32),
            pltpu.VMEM((2, 2, m_per, kh), jnp.float32),
            pltpu.VMEM((1, 128), jnp.float32),
            pltpu.VMEM((N_DEV, 1, 128), jnp.float32),
            pltpu.SemaphoreType.DMA((2, 2)),
            pltpu.SemaphoreType.DMA((2, 2)),
            pltpu.SemaphoreType.REGULAR((2,)),
            pltpu.SemaphoreType.DMA((2, 2)),
            pltpu.SemaphoreType.DMA((2, 2)),
            pltpu.SemaphoreType.REGULAR((2,)),
            pltpu.SemaphoreType.DMA((N_DEV,)),
            pltpu.SemaphoreType.DMA((N_DEV,)),
        ],
        compiler_params=pltpu.CompilerParams(collective_id=0),
    )(ids, x, w_mat)


# device time: 397713 ns/iter; 2.5435x vs baseline; 1.0300x over previous
import numpy as np

import jax
import jax.numpy as jnp
from jax import lax
from jax.experimental import pallas as pl
from jax.experimental.pallas import tpu as pltpu

N_DEV = 32
H_F = 16
H_B = 15


def _snake_logical(x, y, z):
    return z * 8 + y * 2 + (x if y % 2 == 0 else 1 - x)


def _ham_cycle():
    v16 = [(0, 0), (0, 1), (0, 2), (0, 3), (1, 3), (1, 2), (1, 1),
           (2, 1), (2, 2), (2, 3), (3, 3), (3, 2), (3, 1), (3, 0),
           (2, 0), (1, 0)]
    ham = [_snake_logical(0, y, z) for (y, z) in v16]
    ham += [_snake_logical(1, y, z) for (y, z) in reversed(v16)]
    return np.array(ham, np.int32)


_HAM = _ham_cycle()
_POS = np.zeros(N_DEV, np.int32)
_POS[_HAM] = np.arange(N_DEV, dtype=np.int32)


def kernel(x, w_mat):
    m_per, k = x.shape
    _, n_per = w_mat.shape
    m_tot = N_DEV * m_per
    kh = k // 2

    me = lax.axis_index("i")
    ham = jnp.asarray(_HAM)
    p = jnp.asarray(_POS)[me]
    nxt = ham[(p + 1) % N_DEV]
    prv = ham[(p - 1) % N_DEV]
    fwd_origins = ham[(p - 1 - jnp.arange(H_F)) % N_DEV]
    bwd_origins = ham[(p + 1 + jnp.arange(H_B)) % N_DEV]
    ids = jnp.concatenate(
        [jnp.stack([nxt, prv]), fwd_origins, bwd_origins]).astype(jnp.int32)

    def body(ids_ref, x_ref, w_ref, out_ref, fcomm, bcomm, asend_ref,
             arecv_ref, fsend_sems, frecv_sems, fcredit, bsend_sems,
             brecv_sems, bcredit, absend_sems, abrecv_sems):
        me = lax.axis_index("i")
        nxt = ids_ref[0]
        prv = ids_ref[1]

        def gemm_store2(sub0, sub1, origin):
            blk = (jnp.dot(sub0, w_ref[0:kh, :],
                           preferred_element_type=jnp.float32,
                           precision=lax.Precision.HIGHEST)
                   + jnp.dot(sub1, w_ref[kh:k, :],
                             preferred_element_type=jnp.float32,
                             precision=lax.Precision.HIGHEST))
            blk = jnp.maximum(blk, 0.0)
            out_ref[pl.ds(origin * m_per, m_per), :] = blk
            return jnp.max(blk)

        def fdesc(h, j):
            src = (x_ref.at[:, pl.ds(j * kh, kh)] if h == 0
                   else fcomm.at[(h - 1) % 2, j])
            return pltpu.make_async_remote_copy(
                src_ref=src, dst_ref=fcomm.at[h % 2, j],
                send_sem=fsend_sems.at[h % 2, j],
                recv_sem=frecv_sems.at[h % 2, j],
                device_id=(nxt,), device_id_type=pl.DeviceIdType.MESH)

        def bdesc(h, j):
            src = (x_ref.at[:, pl.ds(j * kh, kh)] if h == 0
                   else bcomm.at[(h - 1) % 2, j])
            return pltpu.make_async_remote_copy(
                src_ref=src, dst_ref=bcomm.at[h % 2, j],
                send_sem=bsend_sems.at[h % 2, j],
                recv_sem=brecv_sems.at[h % 2, j],
                device_id=(prv,), device_id_type=pl.DeviceIdType.MESH)

        barrier_sem = pltpu.get_barrier_semaphore()
        pl.semaphore_signal(barrier_sem, inc=1, device_id=(prv,),
                            device_id_type=pl.DeviceIdType.MESH)
        pl.semaphore_signal(barrier_sem, inc=1, device_id=(nxt,),
                            device_id_type=pl.DeviceIdType.MESH)
        pl.semaphore_wait(barrier_sem, 2)

        starts0 = [fdesc(0, 0), fdesc(0, 1), bdesc(0, 0), bdesc(0, 1)]
        for d in starts0:
            d.start()
        amax = gemm_store2(x_ref[:, 0:kh], x_ref[:, kh:k], me)
        for d in starts0:
            d.wait_send()

        for h in range(1, 16):
            sp = (h - 1) % 2
            s = h % 2
            fr0 = fdesc(h - 1, 0)
            fr0.wait_recv()
            if h >= 2:
                pl.semaphore_wait(fcredit.at[s], 1)
            fh0 = fdesc(h, 0)
            fh0.start()
            br0 = bdesc(h - 1, 0)
            br0.wait_recv()
            if h >= 2:
                pl.semaphore_wait(bcredit.at[s], 1)
            if h < 15:
                bh0 = bdesc(h, 0)
                bh0.start()
            fr1 = fdesc(h - 1, 1)
            fr1.wait_recv()
            if h < 15:
                fh1 = fdesc(h, 1)
                fh1.start()
            br1 = bdesc(h - 1, 1)
            br1.wait_recv()
            bh1 = bdesc(h, 1)
            bh1.start()

            amax = jnp.maximum(amax, gemm_store2(
                fcomm[sp, 0], fcomm[sp, 1], ids_ref[2 + (h - 1)]))
            amax = jnp.maximum(amax, gemm_store2(
                bcomm[sp, 0], bcomm[sp, 1], ids_ref[2 + H_F + (h - 1)]))

            fh0.wait_send()
            if h < 15:
                fh1.wait_send()
                bh0.wait_send()
            bh1.wait_send()
            if h <= 14:
                pl.semaphore_signal(fcredit.at[sp], inc=1,
                                    device_id=(prv,),
                                    device_id_type=pl.DeviceIdType.MESH)
                pl.semaphore_signal(bcredit.at[sp], inc=1,
                                    device_id=(nxt,),
                                    device_id_type=pl.DeviceIdType.MESH)

        fr0 = fdesc(15, 0)
        fr0.wait_recv()
        br1 = bdesc(15, 1)
        br1.wait_recv()
        amax = jnp.maximum(amax, gemm_store2(
            fcomm[1, 0], bcomm[1, 1], ids_ref[2 + 15]))

        asend_ref[...] = jnp.full((1, 128), amax, jnp.float32)
        sends = []
        for jj in range(1, N_DEV):
            tgt = (me + jj) % N_DEV
            d = pltpu.make_async_remote_copy(
                src_ref=asend_ref, dst_ref=arecv_ref.at[me],
                send_sem=absend_sems.at[jj],
                recv_sem=abrecv_sems.at[me],
                device_id=(tgt,), device_id_type=pl.DeviceIdType.MESH)
            d.start()
            sends.append(d)
        arecv_ref[pl.ds(me, 1), :, :] = jnp.full((1, 1, 128), amax,
                                                 jnp.float32)
        for jj in range(1, N_DEV):
            j = (me + jj) % N_DEV
            rw = pltpu.make_async_remote_copy(
                src_ref=asend_ref, dst_ref=arecv_ref.at[j],
                send_sem=absend_sems.at[0], recv_sem=abrecv_sems.at[j],
                device_id=(me,), device_id_type=pl.DeviceIdType.MESH)
            rw.wait_recv()
        for d in sends:
            d.wait_send()
        amax = jnp.max(arecv_ref[...])

        scale = amax / 448.0
        a = jnp.minimum(out_ref[...] / scale, 448.0)
        u = lax.bitcast_convert_type(a, jnp.int32)
        lsb = lax.shift_right_logical(u, 20) & 1
        ur = lax.shift_left(
            lax.shift_right_logical(u + 0x7FFFF + lsb, 20), 20)
        q = lax.bitcast_convert_type(ur, jnp.float32)
        out_ref[...] = q * scale

    return pl.pallas_call(
        body,
        out_shape=jax.ShapeDtypeStruct((m_tot, n_per), jnp.float32),
        in_specs=[pl.BlockSpec(memory_space=pltpu.SMEM),
                  pl.BlockSpec(memory_space=pltpu.VMEM),
                  pl.BlockSpec(memory_space=pltpu.VMEM)],
        out_specs=pl.BlockSpec(memory_space=pltpu.VMEM),
        scratch_shapes=[
            pltpu.VMEM((2, 2, m_per, kh), jnp.float32),
            pltpu.VMEM((2, 2, m_per, kh), jnp.float32),
            pltpu.VMEM((1, 128), jnp.float32),
            pltpu.VMEM((N_DEV, 1, 128), jnp.float32),
            pltpu.SemaphoreType.DMA((2, 2)),
            pltpu.SemaphoreType.DMA((2, 2)),
            pltpu.SemaphoreType.REGULAR((2,)),
            pltpu.SemaphoreType.DMA((2, 2)),
            pltpu.SemaphoreType.DMA((2, 2)),
            pltpu.SemaphoreType.REGULAR((2,)),
            pltpu.SemaphoreType.DMA((N_DEV,)),
            pltpu.SemaphoreType.DMA((N_DEV,)),
        ],
        compiler_params=pltpu.CompilerParams(collective_id=0),
    )(ids, x, w_mat)


# device time: 397486 ns/iter; 2.5450x vs baseline; 1.0006x over previous
import numpy as np

import jax
import jax.numpy as jnp
from jax import lax
from jax.experimental import pallas as pl
from jax.experimental.pallas import tpu as pltpu

N_DEV = 32
H_F = 16
H_B = 15


def _snake_logical(x, y, z):
    return z * 8 + y * 2 + (x if y % 2 == 0 else 1 - x)


def _ham_cycle():
    v16 = [(0, 0), (0, 1), (0, 2), (0, 3), (1, 3), (1, 2), (1, 1),
           (2, 1), (2, 2), (2, 3), (3, 3), (3, 2), (3, 1), (3, 0),
           (2, 0), (1, 0)]
    ham = [_snake_logical(0, y, z) for (y, z) in v16]
    ham += [_snake_logical(1, y, z) for (y, z) in reversed(v16)]
    return np.array(ham, np.int32)


_HAM = _ham_cycle()
_POS = np.zeros(N_DEV, np.int32)
_POS[_HAM] = np.arange(N_DEV, dtype=np.int32)


def kernel(x, w_mat):
    m_per, k = x.shape
    _, n_per = w_mat.shape
    m_tot = N_DEV * m_per
    kh = k // 2

    me = lax.axis_index("i")
    ham = jnp.asarray(_HAM)
    p = jnp.asarray(_POS)[me]
    nxt = ham[(p + 1) % N_DEV]
    prv = ham[(p - 1) % N_DEV]
    fwd_origins = ham[(p - 1 - jnp.arange(H_F)) % N_DEV]
    bwd_origins = ham[(p + 1 + jnp.arange(H_B)) % N_DEV]
    ids = jnp.concatenate(
        [jnp.stack([nxt, prv]), fwd_origins, bwd_origins]).astype(jnp.int32)

    def body(ids_ref, x_ref, w_ref, out_ref, fcomm, bcomm, asend_ref,
             arecv_ref, fsend_sems, frecv_sems, fcredit, bsend_sems,
             brecv_sems, bcredit, absend_sems, abrecv_sems):
        me = lax.axis_index("i")
        nxt = ids_ref[0]
        prv = ids_ref[1]

        def gemm_store2(sub0, sub1, origin):
            blk = (jnp.dot(sub0, w_ref[0:kh, :],
                           preferred_element_type=jnp.float32,
                           precision=lax.Precision.HIGHEST)
                   + jnp.dot(sub1, w_ref[kh:k, :],
                             preferred_element_type=jnp.float32,
                             precision=lax.Precision.HIGHEST))
            blk = jnp.maximum(blk, 0.0)
            out_ref[pl.ds(origin * m_per, m_per), :] = blk
            return jnp.max(blk)

        def fdesc(h, j):
            src = (x_ref.at[:, pl.ds(j * kh, kh)] if h == 0
                   else fcomm.at[(h - 1) % 2, j])
            return pltpu.make_async_remote_copy(
                src_ref=src, dst_ref=fcomm.at[h % 2, j],
                send_sem=fsend_sems.at[h % 2, j],
                recv_sem=frecv_sems.at[h % 2, j],
                device_id=(nxt,), device_id_type=pl.DeviceIdType.MESH)

        def bdesc(h, j):
            src = (x_ref.at[:, pl.ds(j * kh, kh)] if h == 0
                   else bcomm.at[(h - 1) % 2, j])
            return pltpu.make_async_remote_copy(
                src_ref=src, dst_ref=bcomm.at[h % 2, j],
                send_sem=bsend_sems.at[h % 2, j],
                recv_sem=brecv_sems.at[h % 2, j],
                device_id=(prv,), device_id_type=pl.DeviceIdType.MESH)

        barrier_sem = pltpu.get_barrier_semaphore()
        pl.semaphore_signal(barrier_sem, inc=1, device_id=(prv,),
                            device_id_type=pl.DeviceIdType.MESH)
        pl.semaphore_signal(barrier_sem, inc=1, device_id=(nxt,),
                            device_id_type=pl.DeviceIdType.MESH)
        pl.semaphore_wait(barrier_sem, 2)

        starts0 = [fdesc(0, 0), fdesc(0, 1), bdesc(0, 0), bdesc(0, 1)]
        for d in starts0:
            d.start()
        amax = gemm_store2(x_ref[:, 0:kh], x_ref[:, kh:k], me)
        for d in starts0:
            d.wait_send()


        fr0 = fdesc(0, 0)
        fr0.wait_recv()
        fh0 = fdesc(1, 0)
        fh0.start()
        br0 = bdesc(0, 0)
        br0.wait_recv()
        bh0 = bdesc(1, 0)
        bh0.start()
        fr1 = fdesc(0, 1)
        fr1.wait_recv()
        fh1 = fdesc(1, 1)
        fh1.start()
        br1 = bdesc(0, 1)
        br1.wait_recv()
        bh1 = bdesc(1, 1)
        bh1.start()
        amax = jnp.maximum(amax, gemm_store2(
            fcomm[0, 0], fcomm[0, 1], ids_ref[2]))
        amax = jnp.maximum(amax, gemm_store2(
            bcomm[0, 0], bcomm[0, 1], ids_ref[2 + H_F]))
        fh0.wait_send()
        fh1.wait_send()
        bh0.wait_send()
        bh1.wait_send()
        pl.semaphore_signal(fcredit.at[0], inc=1, device_id=(prv,),
                            device_id_type=pl.DeviceIdType.MESH)
        pl.semaphore_signal(bcredit.at[0], inc=1, device_id=(nxt,),
                            device_id_type=pl.DeviceIdType.MESH)

        def fdyn(h, j):
            return pltpu.make_async_remote_copy(
                src_ref=fcomm.at[(h - 1) % 2, j], dst_ref=fcomm.at[h % 2, j],
                send_sem=fsend_sems.at[h % 2, j],
                recv_sem=frecv_sems.at[h % 2, j],
                device_id=(nxt,), device_id_type=pl.DeviceIdType.MESH)

        def bdyn(h, j):
            return pltpu.make_async_remote_copy(
                src_ref=bcomm.at[(h - 1) % 2, j], dst_ref=bcomm.at[h % 2, j],
                send_sem=bsend_sems.at[h % 2, j],
                recv_sem=brecv_sems.at[h % 2, j],
                device_id=(prv,), device_id_type=pl.DeviceIdType.MESH)

        def hop_body(h, amax):
            sp = (h - 1) % 2
            s = h % 2
            fr0 = fdyn(h - 1, 0)
            fr0.wait_recv()
            pl.semaphore_wait(fcredit.at[s], 1)
            fh0 = fdyn(h, 0)
            fh0.start()
            br0 = bdyn(h - 1, 0)
            br0.wait_recv()
            pl.semaphore_wait(bcredit.at[s], 1)
            bh0 = bdyn(h, 0)
            bh0.start()
            fr1 = fdyn(h - 1, 1)
            fr1.wait_recv()
            fh1 = fdyn(h, 1)
            fh1.start()
            br1 = bdyn(h - 1, 1)
            br1.wait_recv()
            bh1 = bdyn(h, 1)
            bh1.start()
            amax = jnp.maximum(amax, gemm_store2(
                fcomm[sp, 0], fcomm[sp, 1], ids_ref[h + 1]))
            amax = jnp.maximum(amax, gemm_store2(
                bcomm[sp, 0], bcomm[sp, 1], ids_ref[h + 1 + H_F]))
            fh0.wait_send()
            fh1.wait_send()
            bh0.wait_send()
            bh1.wait_send()
            pl.semaphore_signal(fcredit.at[sp], inc=1, device_id=(prv,),
                                device_id_type=pl.DeviceIdType.MESH)
            pl.semaphore_signal(bcredit.at[sp], inc=1, device_id=(nxt,),
                                device_id_type=pl.DeviceIdType.MESH)
            return amax

        amax = lax.fori_loop(2, 15, hop_body, amax)

        fr0 = fdesc(14, 0)
        fr0.wait_recv()
        pl.semaphore_wait(fcredit.at[1], 1)
        fh0 = fdesc(15, 0)
        fh0.start()
        br0 = bdesc(14, 0)
        br0.wait_recv()
        pl.semaphore_wait(bcredit.at[1], 1)
        fr1 = fdesc(14, 1)
        fr1.wait_recv()
        br1 = bdesc(14, 1)
        br1.wait_recv()
        bh1 = bdesc(15, 1)
        bh1.start()
        amax = jnp.maximum(amax, gemm_store2(
            fcomm[0, 0], fcomm[0, 1], ids_ref[2 + 14]))
        amax = jnp.maximum(amax, gemm_store2(
            bcomm[0, 0], bcomm[0, 1], ids_ref[2 + H_F + 14]))
        fh0.wait_send()
        bh1.wait_send()

        fr0 = fdesc(15, 0)
        fr0.wait_recv()
        br1 = bdesc(15, 1)
        br1.wait_recv()
        amax = jnp.maximum(amax, gemm_store2(
            fcomm[1, 0], bcomm[1, 1], ids_ref[2 + 15]))

        asend_ref[...] = jnp.full((1, 128), amax, jnp.float32)
        sends = []
        for jj in range(1, N_DEV):
            tgt = (me + jj) % N_DEV
            d = pltpu.make_async_remote_copy(
                src_ref=asend_ref, dst_ref=arecv_ref.at[me],
                send_sem=absend_sems.at[jj],
                recv_sem=abrecv_sems.at[me],
                device_id=(tgt,), device_id_type=pl.DeviceIdType.MESH)
            d.start()
            sends.append(d)
        arecv_ref[pl.ds(me, 1), :, :] = jnp.full((1, 1, 128), amax,
                                                 jnp.float32)
        for jj in range(1, N_DEV):
            j = (me + jj) % N_DEV
            rw = pltpu.make_async_remote_copy(
                src_ref=asend_ref, dst_ref=arecv_ref.at[j],
                send_sem=absend_sems.at[0], recv_sem=abrecv_sems.at[j],
                device_id=(me,), device_id_type=pl.DeviceIdType.MESH)
            rw.wait_recv()
        for d in sends:
            d.wait_send()
        amax = jnp.max(arecv_ref[...])

        scale = amax / 448.0
        a = jnp.minimum(out_ref[...] / scale, 448.0)
        u = lax.bitcast_convert_type(a, jnp.int32)
        lsb = lax.shift_right_logical(u, 20) & 1
        ur = lax.shift_left(
            lax.shift_right_logical(u + 0x7FFFF + lsb, 20), 20)
        q = lax.bitcast_convert_type(ur, jnp.float32)
        out_ref[...] = q * scale

    return pl.pallas_call(
        body,
        out_shape=jax.ShapeDtypeStruct((m_tot, n_per), jnp.float32),
        in_specs=[pl.BlockSpec(memory_space=pltpu.SMEM),
                  pl.BlockSpec(memory_space=pltpu.VMEM),
                  pl.BlockSpec(memory_space=pltpu.VMEM)],
        out_specs=pl.BlockSpec(memory_space=pltpu.VMEM),
        scratch_shapes=[
            pltpu.VMEM((2, 2, m_per, kh), jnp.float32),
            pltpu.VMEM((2, 2, m_per, kh), jnp.float32),
            pltpu.VMEM((1, 128), jnp.float32),
            pltpu.VMEM((N_DEV, 1, 128), jnp.float32),
            pltpu.SemaphoreType.DMA((2, 2)),
            pltpu.SemaphoreType.DMA((2, 2)),
            pltpu.SemaphoreType.REGULAR((2,)),
            pltpu.SemaphoreType.DMA((2, 2)),
            pltpu.SemaphoreType.DMA((2, 2)),
            pltpu.SemaphoreType.REGULAR((2,)),
            pltpu.SemaphoreType.DMA((N_DEV,)),
            pltpu.SemaphoreType.DMA((N_DEV,)),
        ],
        compiler_params=pltpu.CompilerParams(collective_id=0),
    )(ids, x, w_mat)
